# baseline (speedup 1.0000x reference)
import functools
import jax
import jax.numpy as jnp
from jax.experimental import pallas as pl

N = 4096
D = 512
TOPK = 32
NEG = float("-inf")


def _topk_norm_body(w_ref, out_ref):
    w = w_ref[...]
    B = w.shape[0]
    w3 = w.reshape(B, 32, 128)
    t = jnp.max(w3, axis=1, keepdims=True)
    tops = [t]
    for _ in range(7):
        w3m = jnp.where(w3 < t, w3, NEG)
        t = jnp.max(w3m, axis=1, keepdims=True)
        tops.append(t)
    g = jnp.concatenate(tops, axis=1)

    def step(_, t):
        gm = jnp.where(g < t, g, NEG)
        return jnp.max(gm, axis=(1, 2), keepdims=True)

    t = jax.lax.fori_loop(
        0, TOPK, step, jnp.full((B, 1, 1), jnp.inf, jnp.float32)
    ).reshape(B, 1)
    wsp = jnp.where(w >= t, w, 0.0)
    rs = jnp.sum(wsp, axis=1, keepdims=True) + 1e-8
    out_ref[...] = wsp / rs


def _vproj_body(h_ref, Wv_ref, bv_ref, out_ref):
    out_ref[...] = (
        jnp.dot(h_ref[...], Wv_ref[...], preferred_element_type=jnp.float32)
        + bv_ref[...]
    )


def _msg_body(wn_ref, V_ref, out_ref):
    out_ref[...] = jnp.dot(wn_ref[...], V_ref[...], preferred_element_type=jnp.float32)


def _gate_body(h_ref, msg_ref, Wo_ref, bo_ref, Wg_ref, bg_ref, out_ref):
    h = h_ref[...]
    msg = msg_ref[...]
    out = jnp.dot(msg, Wo_ref[...], preferred_element_type=jnp.float32) + bo_ref[...]
    alpha = jax.nn.sigmoid(
        jnp.dot(h, Wg_ref[...], preferred_element_type=jnp.float32) + bg_ref[...]
    )
    out_ref[...] = alpha * h + (1.0 - alpha) * out


@jax.jit
def kernel(h, w, Wv0, bv0, Wo0, bo0, Wv1, bv1, Wo1, bo1, Wg, bg):
    BR = 512
    BM = 256

    BT = 256
    w_norm = pl.pallas_call(
        _topk_norm_body,
        grid=(N // BT,),
        in_specs=[pl.BlockSpec((BT, N), lambda i: (i, 0))],
        out_specs=pl.BlockSpec((BT, N), lambda i: (i, 0)),
        out_shape=jax.ShapeDtypeStruct((N, N), jnp.float32),
    )(w)

    vproj = pl.pallas_call(
        _vproj_body,
        grid=(N // BR,),
        in_specs=[
            pl.BlockSpec((BR, D), lambda i: (i, 0)),
            pl.BlockSpec((D, D), lambda i: (0, 0)),
            pl.BlockSpec((1, D), lambda i: (0, 0)),
        ],
        out_specs=pl.BlockSpec((BR, D), lambda i: (i, 0)),
        out_shape=jax.ShapeDtypeStruct((N, D), jnp.float32),
    )

    msg_mm = pl.pallas_call(
        _msg_body,
        grid=(N // BM,),
        in_specs=[
            pl.BlockSpec((BM, N), lambda i: (i, 0)),
            pl.BlockSpec((N, D), lambda i: (0, 0)),
        ],
        out_specs=pl.BlockSpec((BM, D), lambda i: (i, 0)),
        out_shape=jax.ShapeDtypeStruct((N, D), jnp.float32),
    )

    gate = pl.pallas_call(
        _gate_body,
        grid=(N // BR,),
        in_specs=[
            pl.BlockSpec((BR, D), lambda i: (i, 0)),
            pl.BlockSpec((BR, D), lambda i: (i, 0)),
            pl.BlockSpec((D, D), lambda i: (0, 0)),
            pl.BlockSpec((1, D), lambda i: (0, 0)),
            pl.BlockSpec((D, 1), lambda i: (0, 0)),
            pl.BlockSpec((1, 1), lambda i: (0, 0)),
        ],
        out_specs=pl.BlockSpec((BR, D), lambda i: (i, 0)),
        out_shape=jax.ShapeDtypeStruct((N, D), jnp.float32),
    )

    bg2 = bg.reshape(1, 1)
    for (Wv, bv, Wo, bo) in ((Wv0, bv0, Wo0, bo0), (Wv1, bv1, Wo1, bo1)):
        V = vproj(h, Wv, bv.reshape(1, D))
        msg = msg_mm(w_norm, V)
        h = gate(h, msg, Wo, bo.reshape(1, D), Wg, bg2)
    return h


# speedup vs baseline: 1.0278x; 1.0278x over previous
import functools
import jax
import jax.numpy as jnp
from jax.experimental import pallas as pl

N = 4096
D = 512
TOPK = 32
NEG = float("-inf")


def _topk_norm_body(w_ref, out_ref):
    w = w_ref[...]
    B = w.shape[0]
    w3 = w.reshape(B, 32, 128)
    t = jnp.max(w3, axis=1, keepdims=True)
    tops = [t]
    for _ in range(7):
        w3m = jnp.where(w3 < t, w3, NEG)
        t = jnp.max(w3m, axis=1, keepdims=True)
        tops.append(t)
    g = jnp.concatenate(tops, axis=1)

    def step(_, t):
        gm = jnp.where(g < t, g, NEG)
        return jnp.max(gm, axis=(1, 2), keepdims=True)

    t = jax.lax.fori_loop(
        0, TOPK, step, jnp.full((B, 1, 1), jnp.inf, jnp.float32)
    ).reshape(B, 1)
    wsp = jnp.where(w >= t, w, 0.0)
    rs = jnp.sum(wsp, axis=1, keepdims=True) + 1e-8
    out_ref[...] = (wsp / rs).astype(jnp.bfloat16)


def _vproj_body(h_ref, Wv_ref, bv_ref, out_ref):
    out_ref[...] = (
        jnp.dot(h_ref[...], Wv_ref[...], preferred_element_type=jnp.float32)
        + bv_ref[...]
    ).astype(jnp.bfloat16)


def _msg_body(wn_ref, V_ref, out_ref):
    out_ref[...] = jnp.dot(wn_ref[...], V_ref[...], preferred_element_type=jnp.float32)


def _gate_body(h_ref, msg_ref, Wo_ref, bo_ref, Wg_ref, bg_ref, out_ref):
    h = h_ref[...]
    msg = msg_ref[...]
    out = jnp.dot(msg, Wo_ref[...], preferred_element_type=jnp.float32) + bo_ref[...]
    alpha = jax.nn.sigmoid(
        jnp.dot(h, Wg_ref[...], preferred_element_type=jnp.float32) + bg_ref[...]
    )
    out_ref[...] = alpha * h + (1.0 - alpha) * out


@jax.jit
def kernel(h, w, Wv0, bv0, Wo0, bo0, Wv1, bv1, Wo1, bo1, Wg, bg):
    BR = 512
    BM = 256

    BT = 256
    w_norm = pl.pallas_call(
        _topk_norm_body,
        grid=(N // BT,),
        in_specs=[pl.BlockSpec((BT, N), lambda i: (i, 0))],
        out_specs=pl.BlockSpec((BT, N), lambda i: (i, 0)),
        out_shape=jax.ShapeDtypeStruct((N, N), jnp.bfloat16),
    )(w)

    vproj = pl.pallas_call(
        _vproj_body,
        grid=(N // BR,),
        in_specs=[
            pl.BlockSpec((BR, D), lambda i: (i, 0)),
            pl.BlockSpec((D, D), lambda i: (0, 0)),
            pl.BlockSpec((1, D), lambda i: (0, 0)),
        ],
        out_specs=pl.BlockSpec((BR, D), lambda i: (i, 0)),
        out_shape=jax.ShapeDtypeStruct((N, D), jnp.bfloat16),
    )

    msg_mm = pl.pallas_call(
        _msg_body,
        grid=(N // BM,),
        in_specs=[
            pl.BlockSpec((BM, N), lambda i: (i, 0)),
            pl.BlockSpec((N, D), lambda i: (0, 0)),
        ],
        out_specs=pl.BlockSpec((BM, D), lambda i: (i, 0)),
        out_shape=jax.ShapeDtypeStruct((N, D), jnp.float32),
    )

    gate = pl.pallas_call(
        _gate_body,
        grid=(N // BR,),
        in_specs=[
            pl.BlockSpec((BR, D), lambda i: (i, 0)),
            pl.BlockSpec((BR, D), lambda i: (i, 0)),
            pl.BlockSpec((D, D), lambda i: (0, 0)),
            pl.BlockSpec((1, D), lambda i: (0, 0)),
            pl.BlockSpec((D, 1), lambda i: (0, 0)),
            pl.BlockSpec((1, 1), lambda i: (0, 0)),
        ],
        out_specs=pl.BlockSpec((BR, D), lambda i: (i, 0)),
        out_shape=jax.ShapeDtypeStruct((N, D), jnp.float32),
    )

    bg2 = bg.reshape(1, 1)
    for (Wv, bv, Wo, bo) in ((Wv0, bv0, Wo0, bo0), (Wv1, bv1, Wo1, bo1)):
        V = vproj(h, Wv, bv.reshape(1, D))
        msg = msg_mm(w_norm, V)
        h = gate(h, msg, Wo, bo.reshape(1, D), Wg, bg2)
    return h


# speedup vs baseline: 1.3486x; 1.3122x over previous
import functools
import jax
import jax.numpy as jnp
from jax.experimental import pallas as pl

N = 4096
D = 512
TOPK = 32
NEG = float("-inf")


def _topk_norm_body(w_ref, out_ref):
    w = w_ref[...]

    def step(_, t):
        masked = jnp.where(w < t, w, NEG)
        return jnp.max(masked, axis=1, keepdims=True)

    t = jax.lax.fori_loop(0, TOPK, step, jnp.full((w.shape[0], 1), jnp.inf, jnp.float32))
    wsp = jnp.where(w >= t, w, 0.0)
    rs = jnp.sum(wsp, axis=1, keepdims=True) + 1e-8
    out_ref[...] = (wsp / rs).astype(jnp.bfloat16)


def _vproj_body(h_ref, Wv_ref, bv_ref, out_ref):
    out_ref[...] = (
        jnp.dot(h_ref[...], Wv_ref[...], preferred_element_type=jnp.float32)
        + bv_ref[...]
    ).astype(jnp.bfloat16)


def _msg_body(wn_ref, V_ref, out_ref):
    out_ref[...] = jnp.dot(wn_ref[...], V_ref[...], preferred_element_type=jnp.float32)


def _gate_body(h_ref, msg_ref, Wo_ref, bo_ref, Wg_ref, bg_ref, out_ref):
    h = h_ref[...]
    msg = msg_ref[...]
    out = jnp.dot(msg, Wo_ref[...], preferred_element_type=jnp.float32) + bo_ref[...]
    alpha = jax.nn.sigmoid(
        jnp.dot(h, Wg_ref[...], preferred_element_type=jnp.float32) + bg_ref[...]
    )
    out_ref[...] = alpha * h + (1.0 - alpha) * out


@jax.jit
def kernel(h, w, Wv0, bv0, Wo0, bo0, Wv1, bv1, Wo1, bo1, Wg, bg):
    BR = 512
    BM = 256

    BT = 512
    w_norm = pl.pallas_call(
        _topk_norm_body,
        grid=(N // BT,),
        in_specs=[pl.BlockSpec((BT, N), lambda i: (i, 0))],
        out_specs=pl.BlockSpec((BT, N), lambda i: (i, 0)),
        out_shape=jax.ShapeDtypeStruct((N, N), jnp.bfloat16),
    )(w)

    vproj = pl.pallas_call(
        _vproj_body,
        grid=(N // BR,),
        in_specs=[
            pl.BlockSpec((BR, D), lambda i: (i, 0)),
            pl.BlockSpec((D, D), lambda i: (0, 0)),
            pl.BlockSpec((1, D), lambda i: (0, 0)),
        ],
        out_specs=pl.BlockSpec((BR, D), lambda i: (i, 0)),
        out_shape=jax.ShapeDtypeStruct((N, D), jnp.bfloat16),
    )

    msg_mm = pl.pallas_call(
        _msg_body,
        grid=(N // BM,),
        in_specs=[
            pl.BlockSpec((BM, N), lambda i: (i, 0)),
            pl.BlockSpec((N, D), lambda i: (0, 0)),
        ],
        out_specs=pl.BlockSpec((BM, D), lambda i: (i, 0)),
        out_shape=jax.ShapeDtypeStruct((N, D), jnp.float32),
    )

    gate = pl.pallas_call(
        _gate_body,
        grid=(N // BR,),
        in_specs=[
            pl.BlockSpec((BR, D), lambda i: (i, 0)),
            pl.BlockSpec((BR, D), lambda i: (i, 0)),
            pl.BlockSpec((D, D), lambda i: (0, 0)),
            pl.BlockSpec((1, D), lambda i: (0, 0)),
            pl.BlockSpec((D, 1), lambda i: (0, 0)),
            pl.BlockSpec((1, 1), lambda i: (0, 0)),
        ],
        out_specs=pl.BlockSpec((BR, D), lambda i: (i, 0)),
        out_shape=jax.ShapeDtypeStruct((N, D), jnp.float32),
    )

    bg2 = bg.reshape(1, 1)
    for (Wv, bv, Wo, bo) in ((Wv0, bv0, Wo0, bo0), (Wv1, bv1, Wo1, bo1)):
        V = vproj(h, Wv, bv.reshape(1, D))
        msg = msg_mm(w_norm, V)
        h = gate(h, msg, Wo, bo.reshape(1, D), Wg, bg2)
    return h


# speedup vs baseline: 1.6804x; 1.2460x over previous
import functools
import jax
import jax.numpy as jnp
from jax.experimental import pallas as pl

N = 4096
D = 512
TOPK = 32
NEG = float("-inf")


def _topk_norm_body(w_ref, out_ref):
    w = w_ref[...]
    B = w.shape[0]
    xs = [w[:, 128 * j : 128 * (j + 1)] for j in range(32)]
    k = 2
    while k <= 32:
        j = k // 2
        while j >= 1:
            for i in range(32):
                l = i ^ j
                if l > i:
                    a, b = xs[i], xs[l]
                    hi = jnp.maximum(a, b)
                    lo = jnp.minimum(a, b)
                    if (i & k) == 0:
                        xs[i], xs[l] = lo, hi
                    else:
                        xs[i], xs[l] = hi, lo
            j //= 2
        k *= 2
    g = jnp.concatenate(xs[24:32], axis=1)

    def step(_, t):
        masked = jnp.where(g < t, g, NEG)
        return jnp.max(masked, axis=1, keepdims=True)

    t = jax.lax.fori_loop(0, TOPK, step, jnp.full((B, 1), jnp.inf, jnp.float32))
    wsp = jnp.where(w >= t, w, 0.0)
    rs = jnp.sum(wsp, axis=1, keepdims=True) + 1e-8
    out_ref[...] = (wsp / rs).astype(jnp.bfloat16)


def _vproj_body(h_ref, Wv_ref, bv_ref, out_ref):
    out_ref[...] = (
        jnp.dot(h_ref[...], Wv_ref[...], preferred_element_type=jnp.float32)
        + bv_ref[...]
    ).astype(jnp.bfloat16)


def _msg_body(wn_ref, V_ref, out_ref):
    out_ref[...] = jnp.dot(wn_ref[...], V_ref[...], preferred_element_type=jnp.float32)


def _gate_body(h_ref, msg_ref, Wo_ref, bo_ref, Wg_ref, bg_ref, out_ref):
    h = h_ref[...]
    msg = msg_ref[...]
    out = jnp.dot(msg, Wo_ref[...], preferred_element_type=jnp.float32) + bo_ref[...]
    alpha = jax.nn.sigmoid(
        jnp.dot(h, Wg_ref[...], preferred_element_type=jnp.float32) + bg_ref[...]
    )
    out_ref[...] = alpha * h + (1.0 - alpha) * out


@jax.jit
def kernel(h, w, Wv0, bv0, Wo0, bo0, Wv1, bv1, Wo1, bo1, Wg, bg):
    BR = 512
    BM = 256

    BT = 256
    w_norm = pl.pallas_call(
        _topk_norm_body,
        grid=(N // BT,),
        in_specs=[pl.BlockSpec((BT, N), lambda i: (i, 0))],
        out_specs=pl.BlockSpec((BT, N), lambda i: (i, 0)),
        out_shape=jax.ShapeDtypeStruct((N, N), jnp.bfloat16),
    )(w)

    vproj = pl.pallas_call(
        _vproj_body,
        grid=(N // BR,),
        in_specs=[
            pl.BlockSpec((BR, D), lambda i: (i, 0)),
            pl.BlockSpec((D, D), lambda i: (0, 0)),
            pl.BlockSpec((1, D), lambda i: (0, 0)),
        ],
        out_specs=pl.BlockSpec((BR, D), lambda i: (i, 0)),
        out_shape=jax.ShapeDtypeStruct((N, D), jnp.bfloat16),
    )

    msg_mm = pl.pallas_call(
        _msg_body,
        grid=(N // BM,),
        in_specs=[
            pl.BlockSpec((BM, N), lambda i: (i, 0)),
            pl.BlockSpec((N, D), lambda i: (0, 0)),
        ],
        out_specs=pl.BlockSpec((BM, D), lambda i: (i, 0)),
        out_shape=jax.ShapeDtypeStruct((N, D), jnp.float32),
    )

    gate = pl.pallas_call(
        _gate_body,
        grid=(N // BR,),
        in_specs=[
            pl.BlockSpec((BR, D), lambda i: (i, 0)),
            pl.BlockSpec((BR, D), lambda i: (i, 0)),
            pl.BlockSpec((D, D), lambda i: (0, 0)),
            pl.BlockSpec((1, D), lambda i: (0, 0)),
            pl.BlockSpec((D, 1), lambda i: (0, 0)),
            pl.BlockSpec((1, 1), lambda i: (0, 0)),
        ],
        out_specs=pl.BlockSpec((BR, D), lambda i: (i, 0)),
        out_shape=jax.ShapeDtypeStruct((N, D), jnp.float32),
    )

    bg2 = bg.reshape(1, 1)
    for (Wv, bv, Wo, bo) in ((Wv0, bv0, Wo0, bo0), (Wv1, bv1, Wo1, bo1)):
        V = vproj(h, Wv, bv.reshape(1, D))
        msg = msg_mm(w_norm, V)
        h = gate(h, msg, Wo, bo.reshape(1, D), Wg, bg2)
    return h


# speedup vs baseline: 1.7420x; 1.0367x over previous
import functools
import jax
import jax.numpy as jnp
from jax.experimental import pallas as pl

N = 4096
D = 512
TOPK = 32
NEG = float("-inf")


def _topk_norm_body(w_ref, out_ref):
    w = w_ref[...]
    B = w.shape[0]
    xs = [w[:, 128 * j : 128 * (j + 1)] for j in range(32)]
    k = 2
    while k <= 32:
        j = k // 2
        while j >= 1:
            for i in range(32):
                l = i ^ j
                if l > i:
                    a, b = xs[i], xs[l]
                    hi = jnp.maximum(a, b)
                    lo = jnp.minimum(a, b)
                    if (i & k) == 0:
                        xs[i], xs[l] = lo, hi
                    else:
                        xs[i], xs[l] = hi, lo
            j //= 2
        k *= 2
    rest = xs[24:31][::-1]

    def pop(_, state):
        heads, depth, _ = state
        m = jnp.max(heads, axis=1, keepdims=True)
        hit = heads == m
        depth = depth + hit.astype(jnp.int32)
        nxt = jnp.full_like(heads, NEG)
        for d_i in range(6, -1, -1):
            nxt = jnp.where(depth == d_i + 1, rest[d_i], nxt)
        heads = jnp.where(hit, nxt, heads)
        return (heads, depth, m)

    _, _, t = jax.lax.fori_loop(
        0,
        TOPK,
        pop,
        (xs[31], jnp.zeros((B, 128), jnp.int32), jnp.zeros((B, 1), jnp.float32)),
    )
    wsp = jnp.where(w >= t, w, 0.0)
    rs = jnp.sum(wsp, axis=1, keepdims=True) + 1e-8
    out_ref[...] = (wsp / rs).astype(jnp.bfloat16)


def _vproj_body(h_ref, Wv_ref, bv_ref, out_ref):
    out_ref[...] = (
        jnp.dot(h_ref[...], Wv_ref[...], preferred_element_type=jnp.float32)
        + bv_ref[...]
    ).astype(jnp.bfloat16)


def _msg_body(wn_ref, V_ref, out_ref):
    out_ref[...] = jnp.dot(wn_ref[...], V_ref[...], preferred_element_type=jnp.float32)


def _gate_body(h_ref, msg_ref, Wo_ref, bo_ref, Wg_ref, bg_ref, out_ref):
    h = h_ref[...]
    msg = msg_ref[...]
    out = jnp.dot(msg, Wo_ref[...], preferred_element_type=jnp.float32) + bo_ref[...]
    alpha = jax.nn.sigmoid(
        jnp.dot(h, Wg_ref[...], preferred_element_type=jnp.float32) + bg_ref[...]
    )
    out_ref[...] = alpha * h + (1.0 - alpha) * out


@jax.jit
def kernel(h, w, Wv0, bv0, Wo0, bo0, Wv1, bv1, Wo1, bo1, Wg, bg):
    BR = 512
    BM = 256

    BT = 256
    w_norm = pl.pallas_call(
        _topk_norm_body,
        grid=(N // BT,),
        in_specs=[pl.BlockSpec((BT, N), lambda i: (i, 0))],
        out_specs=pl.BlockSpec((BT, N), lambda i: (i, 0)),
        out_shape=jax.ShapeDtypeStruct((N, N), jnp.bfloat16),
    )(w)

    vproj = pl.pallas_call(
        _vproj_body,
        grid=(N // BR,),
        in_specs=[
            pl.BlockSpec((BR, D), lambda i: (i, 0)),
            pl.BlockSpec((D, D), lambda i: (0, 0)),
            pl.BlockSpec((1, D), lambda i: (0, 0)),
        ],
        out_specs=pl.BlockSpec((BR, D), lambda i: (i, 0)),
        out_shape=jax.ShapeDtypeStruct((N, D), jnp.bfloat16),
    )

    msg_mm = pl.pallas_call(
        _msg_body,
        grid=(N // BM,),
        in_specs=[
            pl.BlockSpec((BM, N), lambda i: (i, 0)),
            pl.BlockSpec((N, D), lambda i: (0, 0)),
        ],
        out_specs=pl.BlockSpec((BM, D), lambda i: (i, 0)),
        out_shape=jax.ShapeDtypeStruct((N, D), jnp.float32),
    )

    gate = pl.pallas_call(
        _gate_body,
        grid=(N // BR,),
        in_specs=[
            pl.BlockSpec((BR, D), lambda i: (i, 0)),
            pl.BlockSpec((BR, D), lambda i: (i, 0)),
            pl.BlockSpec((D, D), lambda i: (0, 0)),
            pl.BlockSpec((1, D), lambda i: (0, 0)),
            pl.BlockSpec((D, 1), lambda i: (0, 0)),
            pl.BlockSpec((1, 1), lambda i: (0, 0)),
        ],
        out_specs=pl.BlockSpec((BR, D), lambda i: (i, 0)),
        out_shape=jax.ShapeDtypeStruct((N, D), jnp.float32),
    )

    bg2 = bg.reshape(1, 1)
    for (Wv, bv, Wo, bo) in ((Wv0, bv0, Wo0, bo0), (Wv1, bv1, Wo1, bo1)):
        V = vproj(h, Wv, bv.reshape(1, D))
        msg = msg_mm(w_norm, V)
        h = gate(h, msg, Wo, bo.reshape(1, D), Wg, bg2)
    return h
